# baseline (speedup 1.0000x reference)
import functools

import jax
import jax.numpy as jnp
from jax.experimental import pallas as pl
from jax.experimental.pallas import tpu as pltpu
from jax.experimental.pallas import tpu_sc as plsc

INP_DIM = 416.0
NMS_THRES = 0.3
TOP_K = 8
N = 5000
NT = 16
E = 320
CH = E // 16
NP = NT * E
BIG = 1e9


def _sc_body(data_hbm, out_hbm, xchg_hbm, data_v, w_ref, rec_v, rl, outb):
    s_id = jax.lax.axis_index("s")
    c_id = jax.lax.axis_index("c")
    pltpu.sync_copy(data_hbm.at[s_id], data_v)

    io = jax.lax.iota(jnp.int32, 16)
    iof = io.astype(jnp.float32)
    ninf = jnp.float32(-jnp.inf)
    base_f = (s_id * E).astype(jnp.float32)

    z = jnp.zeros((16,), jnp.int32)
    rows = jnp.minimum(io, 11)
    perm = jnp.where(io == 0, 6,
           jnp.where(io == 1, 7,
           jnp.where(io == 2, 8,
           jnp.where(io == 3, 9,
           jnp.where(io == 4, 5,
           jnp.where(io == 5, 10, 0))))))

    bv = jnp.full((16,), ninf, jnp.float32)
    bi = jnp.full((16,), BIG, jnp.float32)
    for c in range(CH):
        sl = pl.ds(c * 16, 16)
        cx = data_v[0, sl]
        cy = data_v[1, sl]
        w = data_v[2, sl]
        h = data_v[3, sl]
        s = data_v[4, sl]
        cf = data_v[5, sl]
        x1c = jnp.clip(cx - w * 0.5, 0.0, INP_DIM)
        y1c = jnp.clip(cy - h * 0.5, 0.0, INP_DIM)
        x2c = jnp.clip(cx + w * 0.5, 0.0, INP_DIM)
        y2c = jnp.clip(cy + h * 0.5, 0.0, INP_DIM)
        off = cf * (INP_DIM + 2.0)
        x1 = x1c + off
        y1 = y1c + off
        x2 = x2c + off
        y2 = y2c + off
        area = (x2 - x1 + 1.0) * (y2 - y1 + 1.0)
        gf = base_f + float(c * 16) + iof
        w_ref[0, sl] = x1
        w_ref[1, sl] = y1
        w_ref[2, sl] = x2
        w_ref[3, sl] = y2
        w_ref[4, sl] = area
        w_ref[5, sl] = s
        w_ref[6, sl] = x1c
        w_ref[7, sl] = y1c
        w_ref[8, sl] = x2c
        w_ref[9, sl] = y2c
        w_ref[10, sl] = cf
        w_ref[11, sl] = gf
        better = (s > bv) | ((s == bv) & (gf < bi))
        bv = jnp.where(better, s, bv)
        bi = jnp.where(better, gf, bi)

    for r in range(TOP_K):
        m = jnp.max(bv)
        gidx = jnp.min(jnp.where(bv == m, bi, BIG))
        li = jnp.clip(gidx - base_f, 0.0, float(E - 1)).astype(jnp.int32)
        rec = plsc.load_gather(w_ref, [rows, jnp.broadcast_to(li, (16,))])
        rec_v[...] = rec
        pltpu.sync_copy(rec_v, xchg_hbm.at[r % 2, c_id * 16 + s_id])
        plsc.subcore_barrier()
        pltpu.sync_copy(xchg_hbm.at[r % 2, pl.ds(c_id * 16, 16)], rl)

        mv = plsc.load_gather(rl, [io, z + 5])
        gv = plsc.load_gather(rl, [io, z + 11])
        mm = jnp.max(mv)
        gg = jnp.min(jnp.where(mv == mm, gv, BIG))
        jf = jnp.min(jnp.where((mv == mm) & (gv == gg), iof, BIG))
        bj = jnp.broadcast_to(jnp.clip(jf, 0.0, 15.0).astype(jnp.int32), (16,))
        ovec = plsc.load_gather(rl, [bj, perm])

        @pl.when((c_id == 0) & (s_id == 0))
        def _():
            outb[r] = ovec

        if r < TOP_K - 1:
            qx1 = plsc.load_gather(rl, [bj, z])
            qy1 = plsc.load_gather(rl, [bj, z + 1])
            qx2 = plsc.load_gather(rl, [bj, z + 2])
            qy2 = plsc.load_gather(rl, [bj, z + 3])
            qar = plsc.load_gather(rl, [bj, z + 4])
            bv = jnp.full((16,), ninf, jnp.float32)
            bi = jnp.full((16,), BIG, jnp.float32)
            for c in range(CH):
                sl = pl.ds(c * 16, 16)
                x1 = w_ref[0, sl]
                y1 = w_ref[1, sl]
                x2 = w_ref[2, sl]
                y2 = w_ref[3, sl]
                ar = w_ref[4, sl]
                av = w_ref[5, sl]
                gf = w_ref[11, sl]
                ix1 = jnp.maximum(x1, qx1)
                iy1 = jnp.maximum(y1, qy1)
                ix2 = jnp.minimum(x2, qx2)
                iy2 = jnp.minimum(y2, qy2)
                inter = (jnp.maximum(ix2 - ix1 + 1.0, 0.0)
                         * jnp.maximum(iy2 - iy1 + 1.0, 0.0))
                iou = inter / (ar + qar - inter + 1e-16)
                new_av = jnp.where(iou > NMS_THRES, ninf, av)
                w_ref[5, sl] = new_av
                better = (new_av > bv) | ((new_av == bv) & (gf < bi))
                bv = jnp.where(better, new_av, bv)
                bi = jnp.where(better, gf, bi)

    @pl.when((c_id == 0) & (s_id == 0))
    def _():
        pltpu.sync_copy(outb, out_hbm)


@jax.jit
def _sc_nms(data):
    fn = pl.kernel(
        _sc_body,
        out_type=(jax.ShapeDtypeStruct((TOP_K, 16), jnp.float32),
                  jax.ShapeDtypeStruct((2, 2 * NT, 16), jnp.float32)),
        mesh=plsc.VectorSubcoreMesh(core_axis_name="c", subcore_axis_name="s"),
        compiler_params=pltpu.CompilerParams(needs_layout_passes=False),
        scratch_types=[
            pltpu.VMEM((6, E), jnp.float32),
            pltpu.VMEM((12, E), jnp.float32),
            pltpu.VMEM((16,), jnp.float32),
            pltpu.VMEM((NT, 16), jnp.float32),
            pltpu.VMEM((TOP_K, 16), jnp.float32),
        ],
    )
    out, _ = fn(data)
    return out


def kernel(boxes, scores, idxs):
    pad = NP - N
    cx = jnp.pad(boxes[:, 0], (0, pad))
    cy = jnp.pad(boxes[:, 1], (0, pad))
    w = jnp.pad(boxes[:, 2], (0, pad))
    h = jnp.pad(boxes[:, 3], (0, pad))
    s = jnp.pad(scores, (0, pad), constant_values=-jnp.inf)
    cf = jnp.pad(idxs.astype(jnp.float32), (0, pad))
    data = jnp.stack([cx, cy, w, h, s, cf])
    data = data.reshape(6, NT, E).transpose(1, 0, 2)
    out = _sc_nms(data)
    return out[:, :6]


# speedup vs baseline: 1.0373x; 1.0373x over previous
import functools

import jax
import jax.numpy as jnp
from jax.experimental import pallas as pl
from jax.experimental.pallas import tpu as pltpu
from jax.experimental.pallas import tpu_sc as plsc

INP_DIM = 416.0
NMS_THRES = 0.3
TOP_K = 8
N = 5000
NT = 16
E = 320
CH = E // 16
NP = NT * E
BIG = 1e9


def _sc_body(data_hbm, out_hbm, xchg_hbm, data_v, w_ref, rec_v, rl, outb):
    s_id = jax.lax.axis_index("s")
    c_id = jax.lax.axis_index("c")
    pltpu.sync_copy(data_hbm.at[s_id], data_v)

    io = jax.lax.iota(jnp.int32, 16)
    iof = io.astype(jnp.float32)
    ninf = jnp.float32(-jnp.inf)
    base_f = (s_id * E).astype(jnp.float32)

    z = jnp.zeros((16,), jnp.int32)
    rows = jnp.minimum(io, 11)
    perm = jnp.where(io == 0, 6,
           jnp.where(io == 1, 7,
           jnp.where(io == 2, 8,
           jnp.where(io == 3, 9,
           jnp.where(io == 4, 5,
           jnp.where(io == 5, 10, 0))))))

    bv = jnp.full((16,), ninf, jnp.float32)
    bi = jnp.full((16,), BIG, jnp.float32)
    for c in range(CH):
        sl = pl.ds(c * 16, 16)
        cx = data_v[0, sl]
        cy = data_v[1, sl]
        w = data_v[2, sl]
        h = data_v[3, sl]
        s = data_v[4, sl]
        cf = data_v[5, sl]
        x1c = jnp.clip(cx - w * 0.5, 0.0, INP_DIM)
        y1c = jnp.clip(cy - h * 0.5, 0.0, INP_DIM)
        x2c = jnp.clip(cx + w * 0.5, 0.0, INP_DIM)
        y2c = jnp.clip(cy + h * 0.5, 0.0, INP_DIM)
        off = cf * (INP_DIM + 2.0)
        x1 = x1c + off
        y1 = y1c + off
        x2 = x2c + off
        y2 = y2c + off
        area = (x2 - x1 + 1.0) * (y2 - y1 + 1.0)
        gf = base_f + float(c * 16) + iof
        w_ref[0, sl] = x1
        w_ref[1, sl] = y1
        w_ref[2, sl] = x2
        w_ref[3, sl] = y2
        w_ref[4, sl] = area
        w_ref[5, sl] = s
        w_ref[6, sl] = x1c
        w_ref[7, sl] = y1c
        w_ref[8, sl] = x2c
        w_ref[9, sl] = y2c
        w_ref[10, sl] = cf
        w_ref[11, sl] = gf
        better = (s > bv) | ((s == bv) & (gf < bi))
        bv = jnp.where(better, s, bv)
        bi = jnp.where(better, gf, bi)

    for r in range(TOP_K):
        m = jnp.max(bv)
        gidx = jnp.min(jnp.where(bv == m, bi, BIG))
        li = jnp.clip(gidx - base_f, 0.0, float(E - 1)).astype(jnp.int32)
        rec = plsc.load_gather(w_ref, [rows, jnp.broadcast_to(li, (16,))])
        rec_v[...] = rec
        pltpu.sync_copy(rec_v, xchg_hbm.at[r % 2, c_id * 16 + s_id])
        plsc.subcore_barrier()
        pltpu.sync_copy(xchg_hbm.at[r % 2, pl.ds(c_id * 16, 16)], rl)

        mv = plsc.load_gather(rl, [io, z + 5])
        gv = plsc.load_gather(rl, [io, z + 11])
        mm = jnp.max(mv)
        gg = jnp.min(jnp.where(mv == mm, gv, BIG))
        jf = jnp.min(jnp.where((mv == mm) & (gv == gg), iof, BIG))
        bj = jnp.broadcast_to(jnp.clip(jf, 0.0, 15.0).astype(jnp.int32), (16,))
        ovec = plsc.load_gather(rl, [bj, perm])

        @pl.when((c_id == 0) & (s_id == 0))
        def _():
            outb[r] = ovec

        if r < TOP_K - 1:
            qx1 = plsc.load_gather(rl, [bj, z])
            qy1 = plsc.load_gather(rl, [bj, z + 1])
            qx2 = plsc.load_gather(rl, [bj, z + 2])
            qy2 = plsc.load_gather(rl, [bj, z + 3])
            qar = plsc.load_gather(rl, [bj, z + 4])
            bv = jnp.full((16,), ninf, jnp.float32)
            bi = jnp.full((16,), BIG, jnp.float32)
            for c in range(CH):
                sl = pl.ds(c * 16, 16)
                x1 = w_ref[0, sl]
                y1 = w_ref[1, sl]
                x2 = w_ref[2, sl]
                y2 = w_ref[3, sl]
                ar = w_ref[4, sl]
                av = w_ref[5, sl]
                gf = w_ref[11, sl]
                ix1 = jnp.maximum(x1, qx1)
                iy1 = jnp.maximum(y1, qy1)
                ix2 = jnp.minimum(x2, qx2)
                iy2 = jnp.minimum(y2, qy2)
                inter = (jnp.maximum(ix2 - ix1 + 1.0, 0.0)
                         * jnp.maximum(iy2 - iy1 + 1.0, 0.0))
                iou = inter / (ar + qar - inter + 1e-16)
                new_av = jnp.where(iou > NMS_THRES, ninf, av)
                w_ref[5, sl] = new_av
                better = (new_av > bv) | ((new_av == bv) & (gf < bi))
                bv = jnp.where(better, new_av, bv)
                bi = jnp.where(better, gf, bi)

    @pl.when((c_id == 0) & (s_id == 0))
    def _():
        pltpu.sync_copy(outb, out_hbm)


@jax.jit
def _sc_nms(data):
    fn = pl.kernel(
        _sc_body,
        out_type=(jax.ShapeDtypeStruct((TOP_K, 16), jnp.float32),
                  jax.ShapeDtypeStruct((2, 2 * NT, 16), jnp.float32)),
        mesh=plsc.VectorSubcoreMesh(core_axis_name="c", subcore_axis_name="s",
                                    num_cores=1),
        compiler_params=pltpu.CompilerParams(needs_layout_passes=False),
        scratch_types=[
            pltpu.VMEM((6, E), jnp.float32),
            pltpu.VMEM((12, E), jnp.float32),
            pltpu.VMEM((16,), jnp.float32),
            pltpu.VMEM((NT, 16), jnp.float32),
            pltpu.VMEM((TOP_K, 16), jnp.float32),
        ],
    )
    out, _ = fn(data)
    return out


def kernel(boxes, scores, idxs):
    pad = NP - N
    cx = jnp.pad(boxes[:, 0], (0, pad))
    cy = jnp.pad(boxes[:, 1], (0, pad))
    w = jnp.pad(boxes[:, 2], (0, pad))
    h = jnp.pad(boxes[:, 3], (0, pad))
    s = jnp.pad(scores, (0, pad), constant_values=-jnp.inf)
    cf = jnp.pad(idxs.astype(jnp.float32), (0, pad))
    data = jnp.stack([cx, cy, w, h, s, cf])
    data = data.reshape(6, NT, E).transpose(1, 0, 2)
    out = _sc_nms(data)
    return out[:, :6]
